# baseline (speedup 1.0000x reference)
import functools

import jax
import jax.numpy as jnp
from jax import lax
from jax.experimental import pallas as pl
from jax.experimental.pallas import tpu as pltpu
from jax.experimental.pallas import tpu_sc as plsc

N_CLS = 1000
CTX_LEN = 16
D_MODEL = 512
SEQ = 77
VOCAB = 49408
NSUF = SEQ - 1 - CTX_LEN
SUF_OFF = 8
IDX_W = 72

NUM_CORES = 2
NUM_SUBCORES = 16
NW = NUM_CORES * NUM_SUBCORES
ITERS = (N_CLS + NW - 1) // NW
NBUF = 2


def _detile_body(x_ref, o_ref):
    blk = x_ref.shape[0]
    o_ref[...] = x_ref[...].reshape(blk * 4, 128)


def _detile(x, blk):
    n = x.shape[0]
    return pl.pallas_call(
        _detile_body,
        grid=(n // blk,),
        in_specs=[pl.BlockSpec((blk, D_MODEL), lambda i: (i, 0))],
        out_specs=pl.BlockSpec((blk * 4, 128), lambda i: (i, 0)),
        out_shape=jax.ShapeDtypeStruct((n * 4, 128), jnp.float32),
    )(x)


@functools.partial(
    pl.kernel,
    out_type=jax.ShapeDtypeStruct((N_CLS, SEQ, D_MODEL), jnp.float32),
    mesh=plsc.VectorSubcoreMesh(core_axis_name="c", subcore_axis_name="s"),
    scratch_types=[
        [pltpu.VMEM((IDX_W,), jnp.int32)] * NBUF,
        [pltpu.VMEM((SEQ, D_MODEL), jnp.float32)] * NBUF,
        [pltpu.SemaphoreType.DMA] * NBUF,
        [pltpu.SemaphoreType.DMA] * NBUF,
    ],
    compiler_params=pltpu.CompilerParams(use_tc_tiling_on_sc=False),
)
def _assemble(table_hbm, ctx_hbm, gidx_hbm, out_hbm, idxs, bufs, sis, sos):
    wid = lax.axis_index("s") * NUM_CORES + lax.axis_index("c")

    def in_copies(c, b):
        idx_v, buf = idxs[b], bufs[b]
        return (
            pltpu.make_async_copy(table_hbm.at[idx_v.at[pl.ds(0, 1)]],
                                  buf.at[pl.ds(0, 1)], sis[b]),
            pltpu.make_async_copy(table_hbm.at[idx_v.at[pl.ds(SUF_OFF, NSUF)]],
                                  buf.at[pl.ds(1 + CTX_LEN, NSUF)], sis[b]),
            pltpu.make_async_copy(ctx_hbm.at[pl.ds(c * CTX_LEN, CTX_LEN)],
                                  buf.at[pl.ds(1, CTX_LEN)], sis[b]),
        )

    def issue(j, b):
        c = j * NW + wid

        @pl.when(c < N_CLS)
        def _():
            pltpu.sync_copy(gidx_hbm.at[pl.ds(c * IDX_W, IDX_W)], idxs[b])
            for cp in in_copies(c, b):
                cp.start()

    def finish(j, b):
        c = j * NW + wid

        @pl.when(c < N_CLS)
        def _():
            for cp in in_copies(c, b):
                cp.wait()
            pltpu.make_async_copy(bufs[b], out_hbm.at[c], sos[b]).start()

    def drain_out(j, b):
        c = j * NW + wid

        @pl.when(c < N_CLS)
        def _():
            pltpu.make_async_copy(bufs[b], out_hbm.at[c], sos[b]).wait()

    issue(0, 0)
    issue(1, 1)

    def body(g, carry):
        j0 = g * NBUF
        j1 = j0 + 1
        finish(j0, 0)
        finish(j1, 1)
        drain_out(j0, 0)
        issue(j0 + NBUF, 0)
        drain_out(j1, 1)
        issue(j1 + NBUF, 1)
        return carry

    lax.fori_loop(0, ITERS // NBUF, body, 0)


def kernel(token_embedding, ctx, tokenized_prompts):
    z = jnp.zeros((N_CLS, 1), jnp.int32)
    gidx = jnp.concatenate(
        [
            tokenized_prompts[:, :1],
            jnp.broadcast_to(z, (N_CLS, SUF_OFF - 1)),
            tokenized_prompts[:, 1 + CTX_LEN:],
            jnp.broadcast_to(z, (N_CLS, IDX_W - SUF_OFF - NSUF)),
        ],
        axis=1,
    ).reshape(-1)
    table_lin = _detile(token_embedding, 256).reshape(VOCAB, D_MODEL)
    ctx_lin = _detile(ctx.reshape(N_CLS * CTX_LEN, D_MODEL), 800)
    ctx_lin = ctx_lin.reshape(N_CLS * CTX_LEN, D_MODEL)
    return _assemble(table_lin, ctx_lin, gidx)


# speedup vs baseline: 1.0007x; 1.0007x over previous
import functools

import jax
import jax.numpy as jnp
from jax import lax
from jax.experimental import pallas as pl
from jax.experimental.pallas import tpu as pltpu
from jax.experimental.pallas import tpu_sc as plsc

N_CLS = 1000
CTX_LEN = 16
D_MODEL = 512
SEQ = 77
VOCAB = 49408
NSUF = SEQ - 1 - CTX_LEN
SUF_OFF = 8
IDX_W = 72

NUM_CORES = 2
NUM_SUBCORES = 16
NW = NUM_CORES * NUM_SUBCORES
ITERS = (N_CLS + NW - 1) // NW
NBUF = 2


def _detile_body(x_ref, o_ref):
    blk = x_ref.shape[0]
    o_ref[...] = x_ref[...].reshape(blk * D_MODEL)


def _detile(x, blk):
    n = x.shape[0]
    return pl.pallas_call(
        _detile_body,
        grid=(n // blk,),
        in_specs=[pl.BlockSpec((blk, D_MODEL), lambda i: (i, 0))],
        out_specs=pl.BlockSpec((blk * D_MODEL,), lambda i: (i,)),
        out_shape=jax.ShapeDtypeStruct((n * D_MODEL,), jnp.float32),
    )(x)


@functools.partial(
    pl.kernel,
    out_type=jax.ShapeDtypeStruct((N_CLS, SEQ, D_MODEL), jnp.float32),
    mesh=plsc.VectorSubcoreMesh(core_axis_name="c", subcore_axis_name="s"),
    scratch_types=[
        [pltpu.VMEM((IDX_W,), jnp.int32)] * NBUF,
        [pltpu.VMEM((SEQ, D_MODEL), jnp.float32)] * NBUF,
        [pltpu.SemaphoreType.DMA] * NBUF,
        [pltpu.SemaphoreType.DMA] * NBUF,
    ],
    compiler_params=pltpu.CompilerParams(use_tc_tiling_on_sc=False),
)
def _assemble(table_hbm, ctx_hbm, gidx_hbm, out_hbm, idxs, bufs, sis, sos):
    wid = lax.axis_index("s") * NUM_CORES + lax.axis_index("c")

    def in_copies(c, b):
        idx_v, buf = idxs[b], bufs[b]
        return (
            pltpu.make_async_copy(table_hbm.at[idx_v.at[pl.ds(0, 1)]],
                                  buf.at[pl.ds(0, 1)], sis[b]),
            pltpu.make_async_copy(table_hbm.at[idx_v.at[pl.ds(SUF_OFF, NSUF)]],
                                  buf.at[pl.ds(1 + CTX_LEN, NSUF)], sis[b]),
            pltpu.make_async_copy(ctx_hbm.at[pl.ds(c * CTX_LEN, CTX_LEN)],
                                  buf.at[pl.ds(1, CTX_LEN)], sis[b]),
        )

    def issue(j, b):
        c = j * NW + wid

        @pl.when(c < N_CLS)
        def _():
            pltpu.sync_copy(gidx_hbm.at[pl.ds(c * IDX_W, IDX_W)], idxs[b])
            for cp in in_copies(c, b):
                cp.start()

    def finish(j, b):
        c = j * NW + wid

        @pl.when(c < N_CLS)
        def _():
            for cp in in_copies(c, b):
                cp.wait()
            pltpu.make_async_copy(bufs[b], out_hbm.at[c], sos[b]).start()

    def drain_out(j, b):
        c = j * NW + wid

        @pl.when(c < N_CLS)
        def _():
            pltpu.make_async_copy(bufs[b], out_hbm.at[c], sos[b]).wait()

    issue(0, 0)
    issue(1, 1)

    def body(g, carry):
        j0 = g * NBUF
        j1 = j0 + 1
        finish(j0, 0)
        finish(j1, 1)
        drain_out(j0, 0)
        issue(j0 + NBUF, 0)
        drain_out(j1, 1)
        issue(j1 + NBUF, 1)
        return carry

    lax.fori_loop(0, ITERS // NBUF, body, 0)


def kernel(token_embedding, ctx, tokenized_prompts):
    z = jnp.zeros((N_CLS, 1), jnp.int32)
    gidx = jnp.concatenate(
        [
            tokenized_prompts[:, :1],
            jnp.broadcast_to(z, (N_CLS, SUF_OFF - 1)),
            tokenized_prompts[:, 1 + CTX_LEN:],
            jnp.broadcast_to(z, (N_CLS, IDX_W - SUF_OFF - NSUF)),
        ],
        axis=1,
    ).reshape(-1)
    table_lin = _detile(token_embedding, 256).reshape(VOCAB, D_MODEL)
    ctx_lin = _detile(ctx.reshape(N_CLS * CTX_LEN, D_MODEL), 800)
    ctx_lin = ctx_lin.reshape(N_CLS * CTX_LEN, D_MODEL)
    return _assemble(table_lin, ctx_lin, gidx)


# speedup vs baseline: 1.1349x; 1.1341x over previous
import functools

import jax
import jax.numpy as jnp
from jax import lax
from jax.experimental import pallas as pl
from jax.experimental.pallas import tpu as pltpu
from jax.experimental.pallas import tpu_sc as plsc

N_CLS = 1000
CTX_LEN = 16
D_MODEL = 512
SEQ = 77
NSUF = SEQ - 1 - CTX_LEN
SUF_OFF = 8
TAIL_OFF = 64
NTAIL = 5
IDX_W = 72
LANES = 16

NUM_CORES = 2
NUM_SUBCORES = 16
NW = NUM_CORES * NUM_SUBCORES
ITERS = (N_CLS + NW - 1) // NW
NBUF = 2


@functools.partial(
    pl.kernel,
    out_type=jax.ShapeDtypeStruct((N_CLS, SEQ, D_MODEL), jnp.float32),
    mesh=plsc.VectorSubcoreMesh(core_axis_name="c", subcore_axis_name="s"),
    scratch_types=[
        [pltpu.VMEM((IDX_W,), jnp.int32)] * NBUF,
        [pltpu.VMEM((SEQ, D_MODEL), jnp.float32)] * NBUF,
        [pltpu.VMEM((CTX_LEN, D_MODEL), jnp.float32)] * NBUF,
        [pltpu.VMEM((8, D_MODEL), jnp.float32)] * NBUF,
        [pltpu.SemaphoreType.DMA] * NBUF,
        [pltpu.SemaphoreType.DMA] * NBUF,
    ],
)
def _assemble(table_hbm, ctx_hbm, gidx_hbm, out_hbm, idxs, bufs, cbufs, tbufs,
              sis, sos):
    wid = lax.axis_index("s") * NUM_CORES + lax.axis_index("c")

    def vcopy_rows(dst, dst_row0, src, n_rows):
        def crow(r, carry2):
            for k in range(D_MODEL // LANES):
                sl = pl.ds(k * LANES, LANES)
                dst[dst_row0 + r, sl] = src[r, sl]
            return carry2

        lax.fori_loop(0, n_rows, crow, 0)

    def in_copies(c, b):
        idx_v, buf = idxs[b], bufs[b]
        return (
            pltpu.make_async_copy(table_hbm.at[idx_v.at[pl.ds(0, 8)]],
                                  buf.at[pl.ds(0, 8)], sis[b]),
            pltpu.make_async_copy(table_hbm.at[idx_v.at[pl.ds(SUF_OFF, 56)]],
                                  buf.at[pl.ds(16, 56)], sis[b]),
            pltpu.make_async_copy(table_hbm.at[idx_v.at[pl.ds(TAIL_OFF, 8)]],
                                  tbufs[b], sis[b]),
            pltpu.make_async_copy(ctx_hbm.at[c], cbufs[b], sis[b]),
        )

    def issue(j, b):
        c = j * NW + wid

        @pl.when(c < N_CLS)
        def _():
            pltpu.sync_copy(gidx_hbm.at[pl.ds(c * IDX_W, IDX_W)], idxs[b])
            for cp in in_copies(c, b):
                cp.start()

    def finish(j, b):
        c = j * NW + wid

        @pl.when(c < N_CLS)
        def _():
            for cp in in_copies(c, b):
                cp.wait()
            vcopy_rows(bufs[b], 72, tbufs[b], NTAIL)
            vcopy_rows(bufs[b], 1, cbufs[b], CTX_LEN)
            pltpu.make_async_copy(bufs[b], out_hbm.at[c], sos[b]).start()

    def drain_out(j, b):
        c = j * NW + wid

        @pl.when(c < N_CLS)
        def _():
            pltpu.make_async_copy(bufs[b], out_hbm.at[c], sos[b]).wait()

    issue(0, 0)
    issue(1, 1)

    def body(g, carry):
        j0 = g * NBUF
        j1 = j0 + 1
        finish(j0, 0)
        finish(j1, 1)
        drain_out(j0, 0)
        issue(j0 + NBUF, 0)
        drain_out(j1, 1)
        issue(j1 + NBUF, 1)
        return carry

    lax.fori_loop(0, ITERS // NBUF, body, 0)


def kernel(token_embedding, ctx, tokenized_prompts):
    z = jnp.zeros((N_CLS, 1), jnp.int32)
    gidx = jnp.concatenate(
        [
            tokenized_prompts[:, :1],
            jnp.broadcast_to(z, (N_CLS, 7)),
            z,
            tokenized_prompts[:, 1 + CTX_LEN:72],
            tokenized_prompts[:, 72:],
            jnp.broadcast_to(z, (N_CLS, 3)),
        ],
        axis=1,
    ).reshape(-1)
    return _assemble(token_embedding, ctx, gidx)
